# Initial kernel scaffold; baseline (speedup 1.0000x reference)
import jax
import jax.numpy as jnp
from jax.experimental import pallas as pl


def kernel(point_map, depth, mask, feats):
    raise NotImplementedError("write your pallas kernel here")


# speedup vs baseline: 16.5711x; 16.5711x over previous
import functools

import jax
import jax.numpy as jnp
from jax import lax
from jax.experimental import pallas as pl
from jax.experimental.pallas import tpu as pltpu
from jax.experimental.pallas import tpu_sc as plsc

_B, _N, _D = 4, 4096, 768
_S = 1024
_SUB = 8
_LANE = _N // _SUB

_NC, _NS = 2, 16
_NW = _NC * _NS
_ROWS = _B * _S
_RPW = _ROWS // _NW


def _fps_body(xyz_ref, seed_ref, inds_ref, px_ref, py_ref, pz_ref):
    xs = xyz_ref[0]
    ys = xyz_ref[1]
    zs = xyz_ref[2]
    i_s = lax.broadcasted_iota(jnp.int32, (_B, _SUB, _LANE), 1)
    i_l = lax.broadcasted_iota(jnp.int32, (_B, _SUB, _LANE), 2)
    iota = i_s * _LANE + i_l
    lane_iota = lax.broadcasted_iota(jnp.int32, (_B, _S), 1)
    boffs = lax.broadcasted_iota(jnp.int32, (_B, _S), 0) * _N

    def step(s, dists):
        m = jnp.max(jnp.max(dists, axis=2, keepdims=True), axis=1, keepdims=True)
        elig = dists == m
        idx = jnp.min(
            jnp.min(jnp.where(elig, iota, _N), axis=2, keepdims=True),
            axis=1, keepdims=True)
        oneh = iota == idx
        zero = jnp.float32(0.0)
        fx = jnp.sum(jnp.sum(jnp.where(oneh, xs, zero), axis=2, keepdims=True),
                     axis=1, keepdims=True)
        fy = jnp.sum(jnp.sum(jnp.where(oneh, ys, zero), axis=2, keepdims=True),
                     axis=1, keepdims=True)
        fz = jnp.sum(jnp.sum(jnp.where(oneh, zs, zero), axis=2, keepdims=True),
                     axis=1, keepdims=True)
        dx = xs - fx
        dy = ys - fy
        dz = zs - fz
        nd = jnp.sqrt(dx * dx + dy * dy + dz * dz + jnp.float32(1e-12))
        gidx = jnp.broadcast_to(idx.reshape(_B, 1), (_B, _S)) + boffs
        wm = lane_iota == s
        inds_ref[...] = jnp.where(wm, gidx, inds_ref[...])
        px_ref[...] = jnp.where(wm, fx.reshape(_B, 1), px_ref[...])
        py_ref[...] = jnp.where(wm, fy.reshape(_B, 1), py_ref[...])
        pz_ref[...] = jnp.where(wm, fz.reshape(_B, 1), pz_ref[...])
        return jnp.minimum(dists, nd)

    lax.fori_loop(0, _S, step, seed_ref[...])


def _run_fps(xyz, seed):
    return pl.pallas_call(
        _fps_body,
        out_shape=(
            jax.ShapeDtypeStruct((_B, _S), jnp.int32),
            jax.ShapeDtypeStruct((_B, _S), jnp.float32),
            jax.ShapeDtypeStruct((_B, _S), jnp.float32),
            jax.ShapeDtypeStruct((_B, _S), jnp.float32),
        ),
    )(xyz, seed)


def _sc_gather_body(feats_hbm, idx_hbm, out_f, idx_v, rows_v, semf):
    wid = lax.axis_index("s") * _NC + lax.axis_index("c")
    base = wid * _RPW
    pltpu.sync_copy(idx_hbm.at[pl.ds(base, _RPW)], idx_v)
    pltpu.async_copy(feats_hbm.at[idx_v], rows_v, semf).wait()
    pltpu.sync_copy(rows_v, out_f.at[pl.ds(base, _RPW)])


@functools.cache
def _sc_gather():
    return pl.kernel(
        _sc_gather_body,
        out_type=jax.ShapeDtypeStruct((_ROWS, _D), jnp.float32),
        mesh=plsc.VectorSubcoreMesh(
            core_axis_name="c", subcore_axis_name="s",
            num_cores=_NC, num_subcores=_NS),
        scratch_types=[
            pltpu.VMEM((_RPW,), jnp.int32),
            pltpu.VMEM((_RPW, _D), jnp.float32),
            pltpu.SemaphoreType.DMA,
        ],
    )


def kernel(point_map, depth, mask, feats):
    mb = mask >= 0.99
    valid_d = jnp.where(mb, depth, jnp.nan)
    median = jnp.nanmedian(valid_d, axis=1, keepdims=True)
    mad = jnp.nanmedian(jnp.abs(valid_d - median), axis=1, keepdims=True)
    new_mask = jnp.abs(depth - median) < 5.0 * mad
    ret = mb & new_mask
    valid_mask = jnp.any(jnp.any(ret, axis=-1, keepdims=True), axis=-2,
                         keepdims=True)
    mb = jnp.where(valid_mask, ret, mb)

    masks = mb.astype(jnp.float32)
    is_empty = ~jnp.any(mb, axis=-1)
    masks = masks.at[:, 0].add(is_empty.astype(jnp.float32))
    logits = jnp.log(lax.stop_gradient(masks))
    init_inds = jax.random.categorical(jax.random.key(42), logits, axis=-1)

    oneh0 = init_inds[:, None] == jnp.arange(_N, dtype=init_inds.dtype)[None, :]
    big = jnp.float32(jnp.finfo(jnp.float32).max)
    seed = jnp.where(oneh0, jnp.float32(jnp.inf),
                     jnp.where(mb, big, jnp.float32(-jnp.inf)))
    seed = seed.reshape(_B, _SUB, _LANE)

    xyz = jnp.transpose(point_map, (2, 0, 1)).reshape(3, _B, _SUB, _LANE)

    inds, px, py, pz = _run_fps(xyz, seed)

    feats2d = feats.reshape(_B * _N, _D)
    out_f = _sc_gather()(feats2d, inds.reshape(_ROWS))

    object_points = jnp.stack([px, py, pz], axis=-1)
    object_feats = out_f.reshape(_B, _S, _D)
    return object_points, object_feats


# speedup vs baseline: 22.0107x; 1.3283x over previous
import functools

import jax
import jax.numpy as jnp
from jax import lax
from jax.experimental import pallas as pl
from jax.experimental.pallas import tpu as pltpu
from jax.experimental.pallas import tpu_sc as plsc

_B, _N, _D = 4, 4096, 768
_S = 1024
_R = 8
_W = _N // 2

_NC, _NS = 2, 16
_NW = _NC * _NS
_ROWS = _B * _S
_RPW = _ROWS // _NW


def _win(da, ia, db, ib):
    return (da > db) | ((da == db) & (ia < ib))


def _fps_body(xyz_ref, seed_ref, pts_ref, inds_ref, px_ref, py_ref, pz_ref):
    xs = xyz_ref[0]
    ys = xyz_ref[1]
    zs = xyz_ref[2]
    sub_i = lax.broadcasted_iota(jnp.int32, (_R, _W), 0)
    lane_i = lax.broadcasted_iota(jnp.int32, (_R, _W), 1)
    idx0 = jnp.where(sub_i >= 4, jnp.int32(_W), jnp.int32(0)) + lane_i
    lane_out = lax.broadcasted_iota(jnp.int32, (_R, _S), 1)
    boffs = (lax.broadcasted_iota(jnp.int32, (_R, _S), 0) & 3) * _N
    revf = (jnp.int32(_N - 1) - idx0).astype(jnp.float32)

    rm = [(sub_i & 3) == b for b in range(3)]

    nch = _W // 128
    rchunks = [revf[:, 128 * k:128 * (k + 1)] for k in range(nch)]

    def _merge(a, b):
        (da, ra), (db, rb) = a, b
        w = (da > db) | ((da == db) & (ra > rb))
        return jnp.where(w, da, db), jnp.where(w, ra, rb)

    def step(s, dists):
        cur = [(dists[:, 128 * k:128 * (k + 1)], rchunks[k])
               for k in range(nch)]
        while len(cur) > 1:
            h = len(cur) // 2
            cur = [_merge(cur[i], cur[i + h]) for i in range(h)]
        d1, r1 = cur[0]
        m = jnp.max(d1, axis=1, keepdims=True)
        rr = jnp.max(jnp.where(d1 == m, r1, jnp.float32(-1.0)),
                     axis=1, keepdims=True)
        fi8 = (jnp.float32(_N - 1) - rr).astype(jnp.int32)
        iscal = []
        for b in range(_B):
            wlo = m[b, 0] >= m[b + 4, 0]
            iscal.append(jnp.where(wlo, fi8[b, 0], fi8[b + 4, 0]))

        def _getc(c):
            sc = [pts_ref[c, b, iscal[b]] for b in range(_B)]
            return jnp.where(rm[0], sc[0],
                             jnp.where(rm[1], sc[1],
                                       jnp.where(rm[2], sc[2], sc[3])))

        fx = _getc(0)
        fy = _getc(1)
        fz = _getc(2)
        fiv = jnp.where(rm[0], iscal[0],
                        jnp.where(rm[1], iscal[1],
                                  jnp.where(rm[2], iscal[2], iscal[3])))
        dx = xs - fx
        dy = ys - fy
        dz = zs - fz
        nd = jnp.sqrt(dx * dx + dy * dy + dz * dz + jnp.float32(1e-12))
        wm = lane_out == s
        inds_ref[...] = jnp.where(wm, fiv[:, :_S] + boffs, inds_ref[...])
        px_ref[...] = jnp.where(wm, fx[:, :_S], px_ref[...])
        py_ref[...] = jnp.where(wm, fy[:, :_S], py_ref[...])
        pz_ref[...] = jnp.where(wm, fz[:, :_S], pz_ref[...])
        return jnp.minimum(dists, nd)

    lax.fori_loop(0, _S, step, seed_ref[...])


def _run_fps(xyz, seed, pts3):
    return pl.pallas_call(
        _fps_body,
        in_specs=[
            pl.BlockSpec(memory_space=pltpu.VMEM),
            pl.BlockSpec(memory_space=pltpu.VMEM),
            pl.BlockSpec(memory_space=pltpu.SMEM),
        ],
        out_shape=(
            jax.ShapeDtypeStruct((_R, _S), jnp.int32),
            jax.ShapeDtypeStruct((_R, _S), jnp.float32),
            jax.ShapeDtypeStruct((_R, _S), jnp.float32),
            jax.ShapeDtypeStruct((_R, _S), jnp.float32),
        ),
    )(xyz, seed, pts3)


def _sc_gather_body(feats_hbm, idx_hbm, out_f, idx_v, rows_v, semf):
    wid = lax.axis_index("s") * _NC + lax.axis_index("c")
    base = wid * _RPW
    pltpu.sync_copy(idx_hbm.at[pl.ds(base, _RPW)], idx_v)
    pltpu.async_copy(feats_hbm.at[idx_v], rows_v, semf).wait()
    pltpu.sync_copy(rows_v, out_f.at[pl.ds(base, _RPW)])


@functools.cache
def _sc_gather():
    return pl.kernel(
        _sc_gather_body,
        out_type=jax.ShapeDtypeStruct((_ROWS, _D), jnp.float32),
        mesh=plsc.VectorSubcoreMesh(
            core_axis_name="c", subcore_axis_name="s",
            num_cores=_NC, num_subcores=_NS),
        scratch_types=[
            pltpu.VMEM((_RPW,), jnp.int32),
            pltpu.VMEM((_RPW, _D), jnp.float32),
            pltpu.SemaphoreType.DMA,
        ],
    )


def _fold(a):
    return a.reshape(_B, 2, _W).transpose(1, 0, 2).reshape(_R, _W)


def kernel(point_map, depth, mask, feats):
    mb = mask >= 0.99
    valid_d = jnp.where(mb, depth, jnp.nan)
    median = jnp.nanmedian(valid_d, axis=1, keepdims=True)
    mad = jnp.nanmedian(jnp.abs(valid_d - median), axis=1, keepdims=True)
    new_mask = jnp.abs(depth - median) < 5.0 * mad
    ret = mb & new_mask
    valid_mask = jnp.any(jnp.any(ret, axis=-1, keepdims=True), axis=-2,
                         keepdims=True)
    mb = jnp.where(valid_mask, ret, mb)

    masks = mb.astype(jnp.float32)
    is_empty = ~jnp.any(mb, axis=-1)
    masks = masks.at[:, 0].add(is_empty.astype(jnp.float32))
    logits = jnp.log(lax.stop_gradient(masks))
    init_inds = jax.random.categorical(jax.random.key(42), logits, axis=-1)

    oneh0 = init_inds[:, None] == jnp.arange(_N, dtype=init_inds.dtype)[None, :]
    big = jnp.float32(jnp.finfo(jnp.float32).max)
    seed = jnp.where(oneh0, jnp.float32(jnp.inf),
                     jnp.where(mb, big, jnp.float32(-jnp.inf)))
    seed = _fold(seed)

    xyz = jnp.stack([_fold(point_map[:, :, 0]),
                     _fold(point_map[:, :, 1]),
                     _fold(point_map[:, :, 2])])

    pts3 = jnp.transpose(point_map, (2, 0, 1))
    inds, px, py, pz = _run_fps(xyz, seed, pts3)

    feats2d = feats.reshape(_B * _N, _D)
    out_f = _sc_gather()(feats2d, inds[:_B].reshape(_ROWS))

    object_points = jnp.stack([px[:_B], py[:_B], pz[:_B]], axis=-1)
    object_feats = out_f.reshape(_B, _S, _D)
    return object_points, object_feats


# speedup vs baseline: 22.4158x; 1.0184x over previous
import functools

import jax
import jax.numpy as jnp
from jax import lax
from jax.experimental import pallas as pl
from jax.experimental.pallas import tpu as pltpu
from jax.experimental.pallas import tpu_sc as plsc

_B, _N, _D = 4, 4096, 768
_S = 1024
_R = 8
_W = _N // 2

_NC, _NS = 2, 16
_NW = _NC * _NS
_ROWS = _B * _S
_RPW = _ROWS // _NW


def _win(da, ia, db, ib):
    return (da > db) | ((da == db) & (ia < ib))


def _fps_body(xyz_ref, seed_ref, pts_ref, inds_ref, px_ref, py_ref, pz_ref):
    xs = xyz_ref[0]
    ys = xyz_ref[1]
    zs = xyz_ref[2]
    sub_i = lax.broadcasted_iota(jnp.int32, (_R, _W), 0)
    lane_i = lax.broadcasted_iota(jnp.int32, (_R, _W), 1)
    idx0 = jnp.where(sub_i >= 4, jnp.int32(_W), jnp.int32(0)) + lane_i
    lane_out = lax.broadcasted_iota(jnp.int32, (_R, _S), 1)
    boffs = (lax.broadcasted_iota(jnp.int32, (_R, _S), 0) & 3) * _N
    revf = (jnp.int32(_N - 1) - idx0).astype(jnp.float32)

    rm = [(sub_i & 3) == b for b in range(3)]

    nch = _W // 128
    rchunks = [revf[:, 128 * k:128 * (k + 1)] for k in range(nch)]

    def _merge(a, b):
        (da, ra), (db, rb) = a, b
        w = (da > db) | ((da == db) & (ra > rb))
        return jnp.where(w, da, db), jnp.where(w, ra, rb)

    def step(s, dists):
        cur = [(dists[:, 128 * k:128 * (k + 1)], rchunks[k])
               for k in range(nch)]
        while len(cur) > 1:
            h = len(cur) // 2
            cur = [_merge(cur[i], cur[i + h]) for i in range(h)]
        d1, r1 = cur[0]
        m = jnp.max(d1, axis=1, keepdims=True)
        rr = jnp.max(jnp.where(d1 == m, r1, jnp.float32(-1.0)),
                     axis=1, keepdims=True)
        fi8 = (jnp.float32(_N - 1) - rr).astype(jnp.int32)
        iscal = []
        for b in range(_B):
            wlo = m[b, 0] >= m[b + 4, 0]
            iscal.append(jnp.where(wlo, fi8[b, 0], fi8[b + 4, 0]))

        def _getc(c):
            sc = [pts_ref[c, b, iscal[b]] for b in range(_B)]
            return jnp.where(rm[0], sc[0],
                             jnp.where(rm[1], sc[1],
                                       jnp.where(rm[2], sc[2], sc[3])))

        fx = _getc(0)
        fy = _getc(1)
        fz = _getc(2)
        fiv = jnp.where(rm[0], iscal[0],
                        jnp.where(rm[1], iscal[1],
                                  jnp.where(rm[2], iscal[2], iscal[3])))
        dx = xs - fx
        dy = ys - fy
        dz = zs - fz
        nd = jnp.sqrt(dx * dx + dy * dy + dz * dz + jnp.float32(1e-12))
        wm = lane_out == s
        inds_ref[...] = jnp.where(wm, fiv[:, :_S] + boffs, inds_ref[...])
        px_ref[...] = jnp.where(wm, fx[:, :_S], px_ref[...])
        py_ref[...] = jnp.where(wm, fy[:, :_S], py_ref[...])
        pz_ref[...] = jnp.where(wm, fz[:, :_S], pz_ref[...])
        return jnp.minimum(dists, nd)

    lax.fori_loop(0, _S, step, seed_ref[...], unroll=8)


def _run_fps(xyz, seed, pts3):
    return pl.pallas_call(
        _fps_body,
        in_specs=[
            pl.BlockSpec(memory_space=pltpu.VMEM),
            pl.BlockSpec(memory_space=pltpu.VMEM),
            pl.BlockSpec(memory_space=pltpu.SMEM),
        ],
        out_shape=(
            jax.ShapeDtypeStruct((_R, _S), jnp.int32),
            jax.ShapeDtypeStruct((_R, _S), jnp.float32),
            jax.ShapeDtypeStruct((_R, _S), jnp.float32),
            jax.ShapeDtypeStruct((_R, _S), jnp.float32),
        ),
    )(xyz, seed, pts3)


def _sc_gather_body(feats_hbm, idx_hbm, out_f, idx_v, rows_v, semf):
    wid = lax.axis_index("s") * _NC + lax.axis_index("c")
    base = wid * _RPW
    pltpu.sync_copy(idx_hbm.at[pl.ds(base, _RPW)], idx_v)
    pltpu.async_copy(feats_hbm.at[idx_v], rows_v, semf).wait()
    pltpu.sync_copy(rows_v, out_f.at[pl.ds(base, _RPW)])


@functools.cache
def _sc_gather():
    return pl.kernel(
        _sc_gather_body,
        out_type=jax.ShapeDtypeStruct((_ROWS, _D), jnp.float32),
        mesh=plsc.VectorSubcoreMesh(
            core_axis_name="c", subcore_axis_name="s",
            num_cores=_NC, num_subcores=_NS),
        scratch_types=[
            pltpu.VMEM((_RPW,), jnp.int32),
            pltpu.VMEM((_RPW, _D), jnp.float32),
            pltpu.SemaphoreType.DMA,
        ],
    )


def _fold(a):
    return a.reshape(_B, 2, _W).transpose(1, 0, 2).reshape(_R, _W)


def kernel(point_map, depth, mask, feats):
    mb = mask >= 0.99
    valid_d = jnp.where(mb, depth, jnp.nan)
    median = jnp.nanmedian(valid_d, axis=1, keepdims=True)
    mad = jnp.nanmedian(jnp.abs(valid_d - median), axis=1, keepdims=True)
    new_mask = jnp.abs(depth - median) < 5.0 * mad
    ret = mb & new_mask
    valid_mask = jnp.any(jnp.any(ret, axis=-1, keepdims=True), axis=-2,
                         keepdims=True)
    mb = jnp.where(valid_mask, ret, mb)

    masks = mb.astype(jnp.float32)
    is_empty = ~jnp.any(mb, axis=-1)
    masks = masks.at[:, 0].add(is_empty.astype(jnp.float32))
    logits = jnp.log(lax.stop_gradient(masks))
    init_inds = jax.random.categorical(jax.random.key(42), logits, axis=-1)

    oneh0 = init_inds[:, None] == jnp.arange(_N, dtype=init_inds.dtype)[None, :]
    big = jnp.float32(jnp.finfo(jnp.float32).max)
    seed = jnp.where(oneh0, jnp.float32(jnp.inf),
                     jnp.where(mb, big, jnp.float32(-jnp.inf)))
    seed = _fold(seed)

    xyz = jnp.stack([_fold(point_map[:, :, 0]),
                     _fold(point_map[:, :, 1]),
                     _fold(point_map[:, :, 2])])

    pts3 = jnp.transpose(point_map, (2, 0, 1))
    inds, px, py, pz = _run_fps(xyz, seed, pts3)

    feats2d = feats.reshape(_B * _N, _D)
    out_f = _sc_gather()(feats2d, inds[:_B].reshape(_ROWS))

    object_points = jnp.stack([px[:_B], py[:_B], pz[:_B]], axis=-1)
    object_feats = out_f.reshape(_B, _S, _D)
    return object_points, object_feats
